# baseline (speedup 1.0000x reference)
import functools
import jax
import jax.numpy as jnp
from jax import lax
from jax.experimental import pallas as pl
from jax.experimental.pallas import tpu as pltpu
from jax.experimental.pallas import tpu_sc as plsc

_NC, _NS = 2, 16
_NW = _NC * _NS
_CH = 16384


def _sc_add_body(x_hbm, t_hbm, o_hbm, t_buf, x_buf):
    wid = lax.axis_index("s") * _NC + lax.axis_index("c")
    f = t_hbm.shape[0]
    rows_per_w = x_hbm.shape[0] // _NW
    n_ch = f // _CH

    pltpu.sync_copy(t_hbm, t_buf)

    def row_body(r, carry):
        row = wid * rows_per_w + r
        for ch in range(n_ch):
            pltpu.sync_copy(x_hbm.at[row, pl.ds(ch * _CH, _CH)], x_buf)

            def add_body(j, c2):
                sl = pl.ds(j * 16, 16)
                x_buf[sl] = x_buf[sl] + t_buf[pl.ds(ch * _CH + j * 16, 16)]
                return c2

            lax.fori_loop(0, _CH // 16, add_body, 0)
            pltpu.sync_copy(x_buf, o_hbm.at[row, pl.ds(ch * _CH, _CH)])
        return carry

    lax.fori_loop(0, rows_per_w, row_body, 0)


def kernel(patches, pos_table):
    b, n, d = patches.shape
    f = n * d
    x_t = jnp.transpose(patches, (0, 2, 1))
    t_t = jnp.transpose(pos_table, (1, 0))
    xv = (x_t.reshape(b, d // 8, 8, n // 128, 128)
          .transpose(0, 1, 3, 2, 4).reshape(b, f))
    tv = (t_t.reshape(d // 8, 8, n // 128, 128)
          .transpose(0, 2, 1, 3).reshape(f))
    sc_add = functools.partial(
        pl.kernel,
        out_type=jax.ShapeDtypeStruct((b, f), patches.dtype),
        mesh=plsc.VectorSubcoreMesh(core_axis_name="c", subcore_axis_name="s"),
        scratch_types=[
            pltpu.VMEM((f,), jnp.float32),
            pltpu.VMEM((_CH,), jnp.float32),
        ],
    )(_sc_add_body)
    out_flat = sc_add(xv, tv)
    out_t = (out_flat.reshape(b, d // 8, n // 128, 8, 128)
             .transpose(0, 1, 3, 2, 4).reshape(b, d, n))
    return jnp.transpose(out_t, (0, 2, 1))


# speedup vs baseline: 8.8154x; 8.8154x over previous
import jax
import jax.numpy as jnp
from jax.experimental import pallas as pl
from jax.experimental.pallas import tpu as pltpu

_BATCH_BLK = 32


def _add_body(x_ref, t_ref, o_ref):
    o_ref[...] = x_ref[...] + t_ref[...]


def kernel(patches, pos_table):
    b, n, d = patches.shape
    x_t = jnp.transpose(patches, (0, 2, 1))
    t_t = jnp.transpose(pos_table, (1, 0))
    out_t = pl.pallas_call(
        _add_body,
        grid=(b // _BATCH_BLK,),
        in_specs=[
            pl.BlockSpec((_BATCH_BLK, d, n), lambda i: (i, 0, 0)),
            pl.BlockSpec((1, d, n), lambda i: (0, 0, 0)),
        ],
        out_specs=pl.BlockSpec((_BATCH_BLK, d, n), lambda i: (i, 0, 0)),
        out_shape=jax.ShapeDtypeStruct((b, d, n), patches.dtype),
        compiler_params=pltpu.CompilerParams(
            dimension_semantics=("arbitrary",),
        ),
    )(x_t, t_t.reshape(1, d, n))
    return jnp.transpose(out_t, (0, 2, 1))
